# baseline (speedup 1.0000x reference)
import functools

import jax
import jax.numpy as jnp
from jax import lax
from jax.experimental import pallas as pl
from jax.experimental.pallas import tpu as pltpu
from jax.experimental.pallas import tpu_sc as plsc

ROWS = 16384
K = 512
M = 128
L = 16

_info = plsc.get_sparse_core_info()
NC = _info.num_cores
NS = _info.num_subcores
NW = NC * NS
ROWS_PER_W = ROWS // NW
R = 64
NCHUNK = ROWS_PER_W // R
GROUPS = M // L


def _sc_body(z_hbm, mask_hbm, out_hbm, mask_v, zbuf, obuf, sem):
    c = lax.axis_index("c")
    s = lax.axis_index("s")
    wid = s * NC + c
    row0 = wid * ROWS_PER_W

    pltpu.sync_copy(mask_hbm, mask_v)
    mask_vecs = [mask_v[pl.ds(L * j, L)] for j in range(GROUPS)]

    def chunk_body(g, _):
        zbase = (row0 + g * R) * K
        pltpu.sync_copy(z_hbm.at[pl.ds(zbase, R * K)], zbuf)

        def row_body(r, _):
            zoff = r * K
            ooff = r * M
            for j in range(GROUPS):
                idx = mask_vecs[j] + zoff
                vals = plsc.load_gather(zbuf, [idx])
                obuf[pl.ds(ooff + L * j, L)] = vals
            return 0

        lax.fori_loop(0, R, row_body, 0)
        pltpu.sync_copy(obuf, out_hbm.at[pl.ds((row0 + g * R) * M, R * M)])
        return 0

    lax.fori_loop(0, NCHUNK, chunk_body, 0)


@jax.jit
def _sc_gather(z_flat, mask):
    mesh = plsc.VectorSubcoreMesh(core_axis_name="c", subcore_axis_name="s")
    kern = functools.partial(
        pl.kernel,
        mesh=mesh,
        compiler_params=pltpu.CompilerParams(needs_layout_passes=False),
        out_type=jax.ShapeDtypeStruct((ROWS * M,), jnp.float32),
        scratch_types=[
            pltpu.VMEM((M,), jnp.int32),
            pltpu.VMEM((R * K,), jnp.float32),
            pltpu.VMEM((R * M,), jnp.float32),
            pltpu.SemaphoreType.DMA,
        ],
    )(_sc_body)
    return kern(z_flat, mask)


def kernel(z, mask):
    z_flat = z.reshape(ROWS * K)
    out_flat = _sc_gather(z_flat, mask.astype(jnp.int32))
    return out_flat.reshape(ROWS, M)


# speedup vs baseline: 1.2289x; 1.2289x over previous
import functools

import jax
import jax.numpy as jnp
from jax import lax
from jax.experimental import pallas as pl
from jax.experimental.pallas import tpu as pltpu
from jax.experimental.pallas import tpu_sc as plsc

ROWS = 16384
K = 512
M = 128
L = 16

_info = plsc.get_sparse_core_info()
NC = _info.num_cores
NS = _info.num_subcores
NW = NC * NS
ROWS_PER_W = ROWS // NW
R = 64
NCHUNK = ROWS_PER_W // R
GROUPS = M // L


def _sc_body(z_hbm, mask_hbm, out_hbm, mask_v,
             zb0, zb1, ob0, ob1, si0, si1, so0, so1):
    c = lax.axis_index("c")
    s = lax.axis_index("s")
    wid = s * NC + c
    row0 = wid * ROWS_PER_W

    pltpu.sync_copy(mask_hbm, mask_v)
    mask_vecs = [mask_v[pl.ds(L * j, L)] for j in range(GROUPS)]

    zbs = [zb0, zb1]
    obs = [ob0, ob1]
    sis = [si0, si1]
    sos = [so0, so1]

    def start_in(g):
        zbase = (row0 + g * R) * K
        return pltpu.async_copy(z_hbm.at[pl.ds(zbase, R * K)],
                                zbs[g % 2], sis[g % 2])

    in_h = [None] * NCHUNK
    out_h = [None] * NCHUNK
    in_h[0] = start_in(0)
    for g in range(NCHUNK):
        if g + 1 < NCHUNK:
            in_h[g + 1] = start_in(g + 1)
        in_h[g].wait()
        if g >= 2:
            out_h[g - 2].wait()
        zb = zbs[g % 2]
        ob = obs[g % 2]

        @plsc.parallel_loop(0, R, unroll=2)
        def _(r):
            zoff = r * K
            ooff = r * M
            for j in range(GROUPS):
                vals = plsc.load_gather(zb, [mask_vecs[j] + zoff])
                ob[pl.ds(ooff + L * j, L)] = vals

        out_h[g] = pltpu.async_copy(
            ob, out_hbm.at[pl.ds((row0 + g * R) * M, R * M)], sos[g % 2])

    out_h[NCHUNK - 2].wait()
    out_h[NCHUNK - 1].wait()


@jax.jit
def _sc_gather(z_flat, mask):
    mesh = plsc.VectorSubcoreMesh(core_axis_name="c", subcore_axis_name="s")
    kern = functools.partial(
        pl.kernel,
        mesh=mesh,
        compiler_params=pltpu.CompilerParams(needs_layout_passes=False),
        out_type=jax.ShapeDtypeStruct((ROWS * M,), jnp.float32),
        scratch_types=[
            pltpu.VMEM((M,), jnp.int32),
            pltpu.VMEM((R * K,), jnp.float32),
            pltpu.VMEM((R * K,), jnp.float32),
            pltpu.VMEM((R * M,), jnp.float32),
            pltpu.VMEM((R * M,), jnp.float32),
            pltpu.SemaphoreType.DMA,
            pltpu.SemaphoreType.DMA,
            pltpu.SemaphoreType.DMA,
            pltpu.SemaphoreType.DMA,
        ],
    )(_sc_body)
    return kern(z_flat, mask)


def kernel(z, mask):
    z_flat = z.reshape(ROWS * K)
    out_flat = _sc_gather(z_flat, mask.astype(jnp.int32))
    return out_flat.reshape(ROWS, M)


# speedup vs baseline: 2.0701x; 1.6845x over previous
import functools

import jax
import jax.numpy as jnp
from jax import lax
from jax.experimental import pallas as pl
from jax.experimental.pallas import tpu as pltpu
from jax.experimental.pallas import tpu_sc as plsc

ROWS = 16384
K = 512
M = 128
L = 16

_info = plsc.get_sparse_core_info()
NC = _info.num_cores
NS = _info.num_subcores
NW = NC * NS
ROWS_PER_W = ROWS // NW
R = 64
NCHUNK = ROWS_PER_W // R
GROUPS = M // L


def _sc_body(z_hbm, mask_hbm, out_hbm, mask_v,
             zb0, zb1, ob0, ob1, si0, si1, so0, so1):
    c = lax.axis_index("c")
    s = lax.axis_index("s")
    wid = s * NC + c
    row0 = wid * ROWS_PER_W

    pltpu.sync_copy(mask_hbm, mask_v)
    mask_vecs = [mask_v[pl.ds(L * j, L)] for j in range(GROUPS)]

    zbs = [zb0, zb1]
    obs = [ob0, ob1]
    sis = [si0, si1]
    sos = [so0, so1]

    def start_in(g):
        return pltpu.async_copy(z_hbm.at[pl.ds(row0 + g * R, R)],
                                zbs[g % 2], sis[g % 2])

    in_h = [None] * NCHUNK
    out_h = [None] * NCHUNK
    in_h[0] = start_in(0)
    for g in range(NCHUNK):
        if g + 1 < NCHUNK:
            in_h[g + 1] = start_in(g + 1)
        in_h[g].wait()
        if g >= 2:
            out_h[g - 2].wait()
        zb = zbs[g % 2]
        ob = obs[g % 2]

        @plsc.parallel_loop(0, R, unroll=2)
        def _(r):
            row_vec = jnp.full((L,), r, jnp.int32)
            for j in range(GROUPS):
                vals = plsc.load_gather(zb, [row_vec, mask_vecs[j]])
                ob[r, pl.ds(L * j, L)] = vals

        out_h[g] = pltpu.async_copy(
            ob, out_hbm.at[pl.ds(row0 + g * R, R)], sos[g % 2])

    out_h[NCHUNK - 2].wait()
    out_h[NCHUNK - 1].wait()


@jax.jit
def _sc_gather(z, mask):
    mesh = plsc.VectorSubcoreMesh(core_axis_name="c", subcore_axis_name="s")
    kern = functools.partial(
        pl.kernel,
        mesh=mesh,
        compiler_params=pltpu.CompilerParams(needs_layout_passes=False),
        out_type=jax.ShapeDtypeStruct((ROWS, M), jnp.float32),
        scratch_types=[
            pltpu.VMEM((M,), jnp.int32),
            pltpu.VMEM((R, K), jnp.float32),
            pltpu.VMEM((R, K), jnp.float32),
            pltpu.VMEM((R, M), jnp.float32),
            pltpu.VMEM((R, M), jnp.float32),
            pltpu.SemaphoreType.DMA,
            pltpu.SemaphoreType.DMA,
            pltpu.SemaphoreType.DMA,
            pltpu.SemaphoreType.DMA,
        ],
    )(_sc_body)
    return kern(z, mask)


def kernel(z, mask):
    return _sc_gather(z, mask.astype(jnp.int32))
